# baseline (speedup 1.0000x reference)
import jax
import jax.numpy as jnp
import numpy as np
from jax import lax
from jax.experimental import pallas as pl
from jax.experimental.pallas import tpu as pltpu
from jax.experimental.pallas import tpu_sc as plsc

ITEM_NUM = 1000000
EMBED = 96
FEATURE_GROUPS = [20, 20, 10, 10, 2, 2, 2, 1, 1, 1]
B = 1024
F = sum(FEATURE_GROUPS)
FPAD = 72
FC = 12
NF = FPAD // FC

NC, NS = 2, 16
NW = NC * NS
U_ROWS = FPAD * B
U_PER_TILE = U_ROWS // NW
CHUNK = 128
NCHUNK = U_PER_TILE // CHUNK
L_PER_TILE = B // NW


def _sc_gather_body(idx_u_hbm, idx_l_hbm, table_hbm, out_u_hbm, out_l_hbm,
                    idx_v, rows_v, lidx_v, lrows_v, sem):
  wid = lax.axis_index("s") * NC + lax.axis_index("c")

  def chunk_step(c, _):
    pltpu.sync_copy(idx_u_hbm.at[wid * NCHUNK + c], idx_v)
    pltpu.async_copy(table_hbm.at[idx_v], rows_v, sem).wait()
    pltpu.sync_copy(rows_v, out_u_hbm.at[pl.ds((wid * NCHUNK + c) * CHUNK, CHUNK)])
    return 0

  lax.fori_loop(0, NCHUNK, chunk_step, 0)

  pltpu.sync_copy(idx_l_hbm.at[wid], lidx_v)
  pltpu.async_copy(table_hbm.at[lidx_v], lrows_v, sem).wait()
  pltpu.sync_copy(lrows_v, out_l_hbm.at[pl.ds(wid * L_PER_TILE, L_PER_TILE)])


def _make_sc_gather():
  return pl.kernel(
      _sc_gather_body,
      out_type=[
          jax.ShapeDtypeStruct((U_ROWS, EMBED), jnp.float32),
          jax.ShapeDtypeStruct((B, EMBED), jnp.float32),
      ],
      mesh=plsc.VectorSubcoreMesh(core_axis_name="c", subcore_axis_name="s",
                                  num_cores=NC, num_subcores=NS),
      scratch_types=[
          pltpu.VMEM((CHUNK,), jnp.int32),
          pltpu.VMEM((CHUNK, EMBED), jnp.float32),
          pltpu.VMEM((L_PER_TILE,), jnp.int32),
          pltpu.VMEM((L_PER_TILE, EMBED), jnp.float32),
          pltpu.SemaphoreType.DMA,
      ],
      compiler_params=pltpu.CompilerParams(use_tc_tiling_on_sc=False),
  )


def _sc_gather(idx_u, idx_l, table):
  return _make_sc_gather()(idx_u, idx_l, table)

_BN_S = 1.0 / np.sqrt(1.0 + 1e-05)


def _dice(x, alpha):
  xp = jax.nn.sigmoid(x * (1.0 / np.sqrt(1.0 + 1e-09)))
  return alpha * (1.0 - xp) * x + xp * x


def _tc_body(u_ref, q_ref, wrep_ref, a_ref, bm_ref, c_ref, b1_ref, al1_ref,
             w2_ref, b2_ref, w3_ref, b3_ref, wf1b_ref, bf1_ref, af1_ref,
             wf2_ref, bf2_ref, af2_ref, wf3_ref, bf3_ref, out_ref, acc_ref):
  i = pl.program_id(0)

  @pl.when(i == 0)
  def _():
    acc_ref[...] = jnp.zeros_like(acc_ref)

  q = q_ref[...]
  u = u_ref[...]
  hq = jnp.dot(q, a_ref[...], preferred_element_type=jnp.float32) + b1_ref[...]
  qrep = jnp.broadcast_to(q[None], (FC, B, EMBED)).reshape(FC * B, EMBED)
  h = (jnp.dot(u, bm_ref[...], preferred_element_type=jnp.float32)
       + jnp.dot(qrep * u, c_ref[...], preferred_element_type=jnp.float32)
       + jnp.broadcast_to(hq[None], (FC, B, 64)).reshape(FC * B, 64))
  h = _dice(h, al1_ref[...])
  h = jnp.dot(h, w2_ref[...], preferred_element_type=jnp.float32) + b2_ref[...]
  att = jnp.dot(h, w3_ref[...], preferred_element_type=jnp.float32) + b3_ref[...]
  pre = u * att

  acc = acc_ref[...]
  for k in range(FC):
    acc = acc + jnp.dot(pre[k * B:(k + 1) * B],
                        wrep_ref[k * EMBED:(k + 1) * EMBED],
                        preferred_element_type=jnp.float32)
  acc_ref[...] = acc

  @pl.when(i == NF - 1)
  def _():
    x1 = (acc_ref[...] + jnp.dot(q, wf1b_ref[...],
                                 preferred_element_type=jnp.float32)
          + bf1_ref[...]) * _BN_S
    h1 = _dice(x1, af1_ref[...])
    h2 = _dice((jnp.dot(h1, wf2_ref[...], preferred_element_type=jnp.float32)
                + bf2_ref[...]) * _BN_S, af2_ref[...])
    out_ref[...] = (jnp.dot(h2, wf3_ref[...], preferred_element_type=jnp.float32)
                    + bf3_ref[...])


def _tc_call(u_t, q, wrep, *ws):
  def full(shape):
    return pl.BlockSpec(shape, lambda i: (0,) * len(shape))
  in_specs = [
      pl.BlockSpec((FC * B, EMBED), lambda i: (i, 0)),
      full((B, EMBED)),
      pl.BlockSpec((FC * EMBED, 200), lambda i: (i, 0)),
  ] + [full(w.shape) for w in ws]
  return pl.pallas_call(
      _tc_body,
      grid=(NF,),
      in_specs=in_specs,
      out_specs=pl.BlockSpec((B, 1), lambda i: (0, 0)),
      out_shape=jax.ShapeDtypeStruct((B, 1), jnp.float32),
      scratch_shapes=[pltpu.VMEM((B, 200), jnp.float32)],
  )(u_t, q, wrep, *ws)


def kernel(batch_user, batch_label, table, W1, b1, alpha1, W2, b2, W3, b3,
           Wf1, bf1, af1, Wf2, bf2, af2, Wf3, bf3):
  batch_user = batch_user.astype(jnp.int32)
  idx_u = jnp.concatenate(
      [batch_user.T, jnp.full((FPAD - F, B), ITEM_NUM, jnp.int32)], axis=0)
  idx_u = idx_u.reshape(NW * NCHUNK, CHUNK)
  idx_l = batch_label.astype(jnp.int32).reshape(NW, L_PER_TILE)

  u_t, q = _sc_gather(idx_u, idx_l, table)

  A = W1[0:EMBED] + W1[2 * EMBED:3 * EMBED]
  Bm = W1[EMBED:2 * EMBED] - W1[2 * EMBED:3 * EMBED]
  C = W1[3 * EMBED:4 * EMBED]
  gof = np.repeat(np.arange(10), FEATURE_GROUPS)
  gof = np.concatenate([gof, np.full(FPAD - F, 9)]).astype(np.int32)
  wrep = Wf1[:960].reshape(10, EMBED, 200)[gof].reshape(FPAD * EMBED, 200)
  wf1b = Wf1[960:]

  return _tc_call(u_t, q, wrep, A, Bm, C, b1[None], alpha1[None], W2,
                  b2[None], W3, b3[None], wf1b, bf1[None], af1[None], Wf2,
                  bf2[None], af2[None], Wf3, bf3[None])


# speedup vs baseline: 1.1793x; 1.1793x over previous
import jax
import jax.numpy as jnp
import numpy as np
from jax import lax
from jax.experimental import pallas as pl
from jax.experimental.pallas import tpu as pltpu
from jax.experimental.pallas import tpu_sc as plsc

ITEM_NUM = 1000000
EMBED = 96
EMB_P = 128
FEATURE_GROUPS = [20, 20, 10, 10, 2, 2, 2, 1, 1, 1]
B = 1024
F = sum(FEATURE_GROUPS)
FPAD = 72
FC = 12
NF = FPAD // FC

NC, NS = 2, 16
NW = NC * NS
U_ROWS = FPAD * B
U_PER_TILE = U_ROWS // NW
CHUNK = 128
NCHUNK = U_PER_TILE // CHUNK
L_PER_TILE = B // NW


def _sc_gather_body(idx_u_hbm, idx_l_hbm, table_hbm, out_u_hbm, out_l_hbm,
                    idx_v, rows_v, lidx_v, lrows_v, sem):
  wid = lax.axis_index("s") * NC + lax.axis_index("c")

  def chunk_step(c, _):
    pltpu.sync_copy(idx_u_hbm.at[wid * NCHUNK + c], idx_v)
    pltpu.async_copy(table_hbm.at[idx_v], rows_v, sem).wait()
    pltpu.sync_copy(rows_v, out_u_hbm.at[pl.ds((wid * NCHUNK + c) * CHUNK, CHUNK)])
    return 0

  lax.fori_loop(0, NCHUNK, chunk_step, 0)

  pltpu.sync_copy(idx_l_hbm.at[wid], lidx_v)
  pltpu.async_copy(table_hbm.at[lidx_v], lrows_v, sem).wait()
  pltpu.sync_copy(lrows_v, out_l_hbm.at[pl.ds(wid * L_PER_TILE, L_PER_TILE)])


def _make_sc_gather():
  return pl.kernel(
      _sc_gather_body,
      out_type=[
          jax.ShapeDtypeStruct((U_ROWS, EMB_P), jnp.float32),
          jax.ShapeDtypeStruct((B, EMB_P), jnp.float32),
      ],
      mesh=plsc.VectorSubcoreMesh(core_axis_name="c", subcore_axis_name="s",
                                  num_cores=NC, num_subcores=NS),
      scratch_types=[
          pltpu.VMEM((CHUNK,), jnp.int32),
          pltpu.VMEM((CHUNK, EMB_P), jnp.float32),
          pltpu.VMEM((L_PER_TILE,), jnp.int32),
          pltpu.VMEM((L_PER_TILE, EMB_P), jnp.float32),
          pltpu.SemaphoreType.DMA,
      ],
  )


def _sc_gather(idx_u, idx_l, table):
  return _make_sc_gather()(idx_u, idx_l, table)


_BN_S = 1.0 / np.sqrt(1.0 + 1e-05)


def _dice(x, alpha):
  xp = jax.nn.sigmoid(x * (1.0 / np.sqrt(1.0 + 1e-09)))
  return alpha * (1.0 - xp) * x + xp * x


def _tc_body(u_ref, q_ref, wrep_ref, a_ref, bm_ref, c_ref, b1_ref, al1_ref,
             w2_ref, b2_ref, w3_ref, b3_ref, wf1b_ref, bf1_ref, af1_ref,
             wf2_ref, bf2_ref, af2_ref, wf3_ref, bf3_ref, out_ref, acc_ref):
  i = pl.program_id(0)

  @pl.when(i == 0)
  def _():
    acc_ref[...] = jnp.zeros_like(acc_ref)

  q = q_ref[...]
  u = u_ref[...]
  hq = jnp.dot(q, a_ref[...], preferred_element_type=jnp.float32) + b1_ref[...]
  qrep = jnp.broadcast_to(q[None], (FC, B, EMB_P)).reshape(FC * B, EMB_P)
  h = (jnp.dot(u, bm_ref[...], preferred_element_type=jnp.float32)
       + jnp.dot(qrep * u, c_ref[...], preferred_element_type=jnp.float32)
       + jnp.broadcast_to(hq[None], (FC, B, 64)).reshape(FC * B, 64))
  h = _dice(h, al1_ref[...])
  h = jnp.dot(h, w2_ref[...], preferred_element_type=jnp.float32) + b2_ref[...]
  att = jnp.dot(h, w3_ref[...], preferred_element_type=jnp.float32) + b3_ref[...]
  pre = u * att

  acc = acc_ref[...]
  for k in range(FC):
    acc = acc + jnp.dot(pre[k * B:(k + 1) * B],
                        wrep_ref[k * EMB_P:(k + 1) * EMB_P],
                        preferred_element_type=jnp.float32)
  acc_ref[...] = acc

  @pl.when(i == NF - 1)
  def _():
    x1 = (acc_ref[...] + jnp.dot(q, wf1b_ref[...],
                                 preferred_element_type=jnp.float32)
          + bf1_ref[...]) * _BN_S
    h1 = _dice(x1, af1_ref[...])
    h2 = _dice((jnp.dot(h1, wf2_ref[...], preferred_element_type=jnp.float32)
                + bf2_ref[...]) * _BN_S, af2_ref[...])
    out_ref[...] = (jnp.dot(h2, wf3_ref[...], preferred_element_type=jnp.float32)
                    + bf3_ref[...])


def _tc_call(u_t, q, wrep, *ws):
  def full(shape):
    return pl.BlockSpec(shape, lambda i: (0,) * len(shape))
  in_specs = [
      pl.BlockSpec((FC * B, EMB_P), lambda i: (i, 0)),
      full((B, EMB_P)),
      pl.BlockSpec((FC * EMB_P, 200), lambda i: (i, 0)),
  ] + [full(w.shape) for w in ws]
  return pl.pallas_call(
      _tc_body,
      grid=(NF,),
      in_specs=in_specs,
      out_specs=pl.BlockSpec((B, 1), lambda i: (0, 0)),
      out_shape=jax.ShapeDtypeStruct((B, 1), jnp.float32),
      scratch_shapes=[pltpu.VMEM((B, 200), jnp.float32)],
  )(u_t, q, wrep, *ws)


def _pad_rows(w, rows=EMB_P - EMBED):
  return jnp.concatenate([w, jnp.zeros((rows, w.shape[1]), w.dtype)], axis=0)


def kernel(batch_user, batch_label, table, W1, b1, alpha1, W2, b2, W3, b3,
           Wf1, bf1, af1, Wf2, bf2, af2, Wf3, bf3):
  batch_user = batch_user.astype(jnp.int32)
  table_p = jnp.pad(table, ((0, 0), (0, EMB_P - EMBED)))

  pad_idx = (jnp.arange((FPAD - F) * B, dtype=jnp.int32) * 997) % ITEM_NUM
  idx_u = jnp.concatenate([batch_user.T.reshape(-1), pad_idx])
  idx_u = idx_u.reshape(NW * NCHUNK, CHUNK)
  idx_l = batch_label.astype(jnp.int32).reshape(NW, L_PER_TILE)

  u_t, q = _sc_gather(idx_u, idx_l, table_p)

  A = _pad_rows(W1[0:EMBED] + W1[2 * EMBED:3 * EMBED])
  Bm = _pad_rows(W1[EMBED:2 * EMBED] - W1[2 * EMBED:3 * EMBED])
  C = _pad_rows(W1[3 * EMBED:4 * EMBED])
  gof = np.repeat(np.arange(10), FEATURE_GROUPS)
  gof = np.concatenate([gof, np.full(FPAD - F, 10)]).astype(np.int32)
  wf1p = jnp.pad(Wf1[:960].reshape(10, EMBED, 200),
                 ((0, 1), (0, EMB_P - EMBED), (0, 0)))
  wrep = wf1p[gof].reshape(FPAD * EMB_P, 200)
  wf1b = _pad_rows(Wf1[960:])

  return _tc_call(u_t, q, wrep, A, Bm, C, b1[None], alpha1[None], W2,
                  b2[None], W3, b3[None], wf1b, bf1[None], af1[None], Wf2,
                  bf2[None], af2[None], Wf3, bf3[None])


# speedup vs baseline: 2.7799x; 2.3572x over previous
import jax
import jax.numpy as jnp
import numpy as np
from jax import lax
from jax.experimental import pallas as pl
from jax.experimental.pallas import tpu as pltpu
from jax.experimental.pallas import tpu_sc as plsc

ITEM_NUM = 1000000
EMBED = 96
EMB_P = 128
FEATURE_GROUPS = [20, 20, 10, 10, 2, 2, 2, 1, 1, 1]
B = 1024
F = sum(FEATURE_GROUPS)
FPAD = 72
FC = 12
NF = FPAD // FC

NC, NS = 2, 16
NW = NC * NS
U_ROWS = FPAD * B
U_PER_TILE = U_ROWS // NW
CHUNK = 128
NCHUNK = U_PER_TILE // CHUNK
L_PER_TILE = B // NW


def _sc_gather_body(idx_u_hbm, idx_l_hbm, table_hbm, out_u_hbm, out_l_hbm,
                    idx_v, rows_v, lidx_v, lrows_v, sem):
  wid = lax.axis_index("s") * NC + lax.axis_index("c")

  def chunk_step(c, _):
    pltpu.sync_copy(idx_u_hbm.at[wid * NCHUNK + c], idx_v)
    pltpu.async_copy(table_hbm.at[idx_v], rows_v, sem).wait()
    pltpu.sync_copy(rows_v, out_u_hbm.at[pl.ds((wid * NCHUNK + c) * CHUNK, CHUNK)])
    return 0

  lax.fori_loop(0, NCHUNK, chunk_step, 0)

  pltpu.sync_copy(idx_l_hbm.at[wid], lidx_v)
  pltpu.async_copy(table_hbm.at[lidx_v], lrows_v, sem).wait()
  pltpu.sync_copy(lrows_v, out_l_hbm.at[pl.ds(wid * L_PER_TILE, L_PER_TILE)])


def _make_sc_gather():
  return pl.kernel(
      _sc_gather_body,
      out_type=[
          jax.ShapeDtypeStruct((U_ROWS, EMB_P), jnp.float32),
          jax.ShapeDtypeStruct((B, EMB_P), jnp.float32),
      ],
      mesh=plsc.VectorSubcoreMesh(core_axis_name="c", subcore_axis_name="s",
                                  num_cores=NC, num_subcores=NS),
      scratch_types=[
          pltpu.VMEM((CHUNK,), jnp.int32),
          pltpu.VMEM((CHUNK, EMB_P), jnp.float32),
          pltpu.VMEM((L_PER_TILE,), jnp.int32),
          pltpu.VMEM((L_PER_TILE, EMB_P), jnp.float32),
          pltpu.SemaphoreType.DMA,
      ],
  )


def _sc_gather(idx_u, idx_l, table):
  return _make_sc_gather()(idx_u, idx_l, table)


_BN_S = 1.0 / np.sqrt(1.0 + 1e-05)


def _dice(x, alpha):
  xp = jax.nn.sigmoid(x * (1.0 / np.sqrt(1.0 + 1e-09)))
  return alpha * (1.0 - xp) * x + xp * x


def _tc_body(u_ref, q_ref, wrep_ref, a_ref, bm_ref, c_ref, b1_ref, al1_ref,
             w2_ref, b2_ref, w3_ref, b3_ref, wf1b_ref, bf1_ref, af1_ref,
             wf2_ref, bf2_ref, af2_ref, wf3_ref, bf3_ref, out_ref, acc_ref):
  i = pl.program_id(0)

  @pl.when(i == 0)
  def _():
    acc_ref[...] = jnp.zeros_like(acc_ref)

  q = q_ref[...]
  u = u_ref[...]
  hq = jnp.dot(q, a_ref[...], preferred_element_type=jnp.float32) + b1_ref[...]
  qrep = jnp.broadcast_to(q[None], (FC, B, EMB_P)).reshape(FC * B, EMB_P)
  h = (jnp.dot(u, bm_ref[...], preferred_element_type=jnp.float32)
       + jnp.dot(qrep * u, c_ref[...], preferred_element_type=jnp.float32)
       + jnp.broadcast_to(hq[None], (FC, B, 64)).reshape(FC * B, 64))
  h = _dice(h, al1_ref[...])
  h = jnp.dot(h, w2_ref[...], preferred_element_type=jnp.float32) + b2_ref[...]
  att = jnp.dot(h, w3_ref[...], preferred_element_type=jnp.float32) + b3_ref[...]
  pre = u * att

  acc = acc_ref[...]
  for k in range(FC):
    acc = acc + jnp.dot(pre[k * B:(k + 1) * B],
                        wrep_ref[k * EMB_P:(k + 1) * EMB_P],
                        preferred_element_type=jnp.float32)
  acc_ref[...] = acc

  @pl.when(i == NF - 1)
  def _():
    x1 = (acc_ref[...] + jnp.dot(q, wf1b_ref[...],
                                 preferred_element_type=jnp.float32)
          + bf1_ref[...]) * _BN_S
    h1 = _dice(x1, af1_ref[...])
    h2 = _dice((jnp.dot(h1, wf2_ref[...], preferred_element_type=jnp.float32)
                + bf2_ref[...]) * _BN_S, af2_ref[...])
    out_ref[...] = (jnp.dot(h2, wf3_ref[...], preferred_element_type=jnp.float32)
                    + bf3_ref[...])


def _tc_call(u_t, q, wrep, *ws):
  def full(shape):
    return pl.BlockSpec(shape, lambda i: (0,) * len(shape))
  in_specs = [
      pl.BlockSpec((FC * B, EMB_P), lambda i: (i, 0)),
      full((B, EMB_P)),
      pl.BlockSpec((FC * EMB_P, 200), lambda i: (i, 0)),
  ] + [full(w.shape) for w in ws]
  return pl.pallas_call(
      _tc_body,
      grid=(NF,),
      in_specs=in_specs,
      out_specs=pl.BlockSpec((B, 1), lambda i: (0, 0)),
      out_shape=jax.ShapeDtypeStruct((B, 1), jnp.float32),
      scratch_shapes=[pltpu.VMEM((B, 200), jnp.float32)],
  )(u_t, q, wrep, *ws)


def _pad_rows(w, rows=EMB_P - EMBED):
  return jnp.concatenate([w, jnp.zeros((rows, w.shape[1]), w.dtype)], axis=0)


_PAD_BLK = 8192


def _pad_body(t_ref, o_ref):
  o_ref[...] = jnp.concatenate(
      [t_ref[...], jnp.zeros((_PAD_BLK, EMB_P - EMBED), jnp.float32)], axis=1)


def _pad_table(table):
  n = table.shape[0]
  grid = (n + _PAD_BLK - 1) // _PAD_BLK
  return pl.pallas_call(
      _pad_body,
      grid=(grid,),
      in_specs=[pl.BlockSpec((_PAD_BLK, EMBED), lambda i: (i, 0))],
      out_specs=pl.BlockSpec((_PAD_BLK, EMB_P), lambda i: (i, 0)),
      out_shape=jax.ShapeDtypeStruct((n, EMB_P), jnp.float32),
  )(table)


def kernel(batch_user, batch_label, table, W1, b1, alpha1, W2, b2, W3, b3,
           Wf1, bf1, af1, Wf2, bf2, af2, Wf3, bf3):
  batch_user = batch_user.astype(jnp.int32)
  table_p = _pad_table(table)

  pad_idx = (jnp.arange((FPAD - F) * B, dtype=jnp.int32) * 997) % ITEM_NUM
  idx_u = jnp.concatenate([batch_user.T.reshape(-1), pad_idx])
  idx_u = idx_u.reshape(NW * NCHUNK, CHUNK)
  idx_l = batch_label.astype(jnp.int32).reshape(NW, L_PER_TILE)

  u_t, q = _sc_gather(idx_u, idx_l, table_p)

  A = _pad_rows(W1[0:EMBED] + W1[2 * EMBED:3 * EMBED])
  Bm = _pad_rows(W1[EMBED:2 * EMBED] - W1[2 * EMBED:3 * EMBED])
  C = _pad_rows(W1[3 * EMBED:4 * EMBED])
  gof = np.repeat(np.arange(10), FEATURE_GROUPS)
  gof = np.concatenate([gof, np.full(FPAD - F, 10)]).astype(np.int32)
  wf1p = jnp.pad(Wf1[:960].reshape(10, EMBED, 200),
                 ((0, 1), (0, EMB_P - EMBED), (0, 0)))
  wrep = wf1p[gof].reshape(FPAD * EMB_P, 200)
  wf1b = _pad_rows(Wf1[960:])

  return _tc_call(u_t, q, wrep, A, Bm, C, b1[None], alpha1[None], W2,
                  b2[None], W3, b3[None], wf1b, bf1[None], af1[None], Wf2,
                  bf2[None], af2[None], Wf3, bf3[None])
